# baseline (speedup 1.0000x reference)
import functools

import jax
import jax.numpy as jnp
from jax import lax
from jax.experimental import pallas as pl
from jax.experimental.pallas import tpu as pltpu
from jax.experimental.pallas import tpu_sc as plsc

D_MODEL = 64
SCALE = float(D_MODEL) ** 0.5
NUM_CORES = 2
NUM_SUBCORES = 16
NUM_WORKERS = NUM_CORES * NUM_SUBCORES
B_TOTAL = 4096 * 200
B_PER_W = B_TOTAL // NUM_WORKERS
CHUNK = 256
NBUF = 4
NCHUNK = B_PER_W // CHUNK
NOUTER = NCHUNK // NBUF


def _emb_body(table_hbm, idx_hbm, out_hbm, idx_v,
              r0, r1, r2, r3, sg0, sg1, sg2, sg3, ss0, ss1, ss2, ss3):
    rows = [r0, r1, r2, r3]
    sg = [sg0, sg1, sg2, sg3]
    ss = [ss0, ss1, ss2, ss3]

    wid = lax.axis_index("s") * NUM_CORES + lax.axis_index("c")
    base = pl.multiple_of(wid * B_PER_W, B_PER_W)
    pltpu.sync_copy(idx_hbm.at[pl.ds(base, B_PER_W)], idx_v)

    def start_gather(ci, b):
        off = pl.multiple_of(ci * CHUNK, CHUNK)
        pltpu.async_copy(
            table_hbm.at[idx_v.at[pl.ds(off, CHUNK)]], rows[b], sg[b])

    def wait_gather(b):
        pltpu.make_async_copy(
            table_hbm.at[idx_v.at[pl.ds(0, CHUNK)]], rows[b], sg[b]).wait()

    def start_store(ci, b):
        off = pl.multiple_of(base + ci * CHUNK, CHUNK)
        pltpu.async_copy(rows[b], out_hbm.at[pl.ds(off, CHUNK)], ss[b])

    def wait_store(b):
        pltpu.make_async_copy(
            rows[b], out_hbm.at[pl.ds(0, CHUNK)], ss[b]).wait()

    def scale(b):
        r = rows[b]

        @plsc.parallel_loop(0, CHUNK, step=1, unroll=8)
        def _(i):
            for j in range(D_MODEL // 16):
                sl = (i, pl.ds(j * 16, 16))
                r[sl] = r[sl] * SCALE

    def stage(ci, b, issue_ahead, wait_ahead_store):
        if issue_ahead:
            bn = (b + NBUF - 1) % NBUF
            if wait_ahead_store:
                wait_store(bn)
            start_gather(ci + NBUF - 1, bn)
        wait_gather(b)
        scale(b)
        start_store(ci, b)

    for b in range(NBUF - 1):
        start_gather(b, b)

    for b in range(NBUF):
        stage(b, b, issue_ahead=True, wait_ahead_store=(b >= 1))

    def outer_body(oi, carry):
        for b in range(NBUF):
            ci = oi * NBUF + b
            stage(ci, b, issue_ahead=True, wait_ahead_store=True)
        return carry

    lax.fori_loop(1, NOUTER - 1, outer_body, 0, unroll=False)

    for b in range(NBUF):
        ci = (NOUTER - 1) * NBUF + b
        stage(ci, b, issue_ahead=(b < 1), wait_ahead_store=True)

    for b in range(NBUF):
        wait_store(b)


@jax.jit
def kernel(data, table):
    b, s = data.shape
    idx = data.reshape(-1).astype(jnp.int32)
    mesh = plsc.VectorSubcoreMesh(
        core_axis_name="c", subcore_axis_name="s", num_cores=NUM_CORES
    )
    gather = pl.kernel(
        _emb_body,
        out_type=jax.ShapeDtypeStruct((b * s, D_MODEL), jnp.float32),
        mesh=mesh,
        scratch_types=(
            [pltpu.VMEM((B_PER_W,), jnp.int32)]
            + [pltpu.VMEM((CHUNK, D_MODEL), jnp.float32) for _ in range(NBUF)]
            + [pltpu.SemaphoreType.DMA for _ in range(2 * NBUF)]
        ),
        compiler_params=pltpu.CompilerParams(use_tc_tiling_on_sc=False),
    )
    out = gather(table, idx)
    return out.reshape(b, s, D_MODEL)
